# baseline (speedup 1.0000x reference)
import jax
import jax.numpy as jnp
from jax.experimental import pallas as pl

HIDDEN = 768
N_EXPERTS = 8
BT = 1024


def _moe_body(x_ref, wr_ref, wet_ref, out_ref):
    x = x_ref[...]
    wr = wr_ref[...]
    logits = jax.lax.dot_general(
        x, wr, (((1,), (1,)), ((), ())),
        preferred_element_type=jnp.float32)
    mx = jnp.max(logits, axis=1, keepdims=True)
    eids = jax.lax.broadcasted_iota(jnp.int32, logits.shape, 1)
    idx = jnp.min(jnp.where(logits == mx, eids, N_EXPERTS), axis=1)

    xb = x.astype(jnp.bfloat16)
    zero = jnp.zeros_like(xb)
    z = jnp.concatenate(
        [jnp.where((idx == e)[:, None], xb, zero) for e in range(N_EXPERTS)],
        axis=1)
    out_ref[...] = jax.lax.dot_general(
        z, wet_ref[...], (((1,), (0,)), ((), ())),
        preferred_element_type=jnp.float32)


@jax.jit
def kernel(x, Wr, We):
    T, H = x.shape
    E = We.shape[0]
    WeT = jnp.transpose(We, (0, 2, 1)).reshape(E * H, H).astype(jnp.bfloat16)
    grid = (T // BT,)
    return pl.pallas_call(
        _moe_body,
        grid=grid,
        in_specs=[
            pl.BlockSpec((BT, H), lambda i: (i, 0)),
            pl.BlockSpec((E, H), lambda i: (0, 0)),
            pl.BlockSpec((E * H, H), lambda i: (0, 0)),
        ],
        out_specs=pl.BlockSpec((BT, H), lambda i: (i, 0)),
        out_shape=jax.ShapeDtypeStruct((T, H), jnp.float32),
    )(x, Wr, WeT)
